# baseline (speedup 1.0000x reference)
import functools

import jax
import jax.numpy as jnp
from jax import lax
from jax.experimental import pallas as pl
from jax.experimental.pallas import tpu as pltpu
from jax.experimental.pallas import tpu_sc as plsc

NUM_FIELDS = 26
VOCAB = 100000
EMBED_DIM = 8
BATCH = 4096

NC, NS = 2, 16
NW = NC * NS
ROWS_PER_W = BATCH // NW
IDX_PER_W = ROWS_PER_W * NUM_FIELDS


def _sc_body(idx_hbm, tab_hbm, out_hbm, idx_v, g_v, s_v, out_v, sem):
    wid = lax.axis_index("s") * NC + lax.axis_index("c")
    lane = lax.iota(jnp.int32, 16)

    pltpu.sync_copy(idx_hbm.at[wid], idx_v)

    def off_body(a, carry):
        base = a * 128
        for q in range(8):
            f = (base + q * 16 + lane) % NUM_FIELDS
            cur = idx_v[a, pl.ds(q * 16, 16)]
            idx_v[a, pl.ds(q * 16, 16)] = cur + f * VOCAB
        return carry

    lax.fori_loop(0, NUM_FIELDS, off_body, 0)

    copies = [
        pltpu.async_copy(
            tab_hbm.at[idx_v.at[g]], g_v.at[pl.ds(g * 128, 128)], sem
        )
        for g in range(NUM_FIELDS)
    ]
    for cp in copies:
        cp.wait()

    rowpat = lax.shift_right_logical(lane, 3)
    colpat = lane % EMBED_DIM

    def red_body(r, carry):
        base = r * NUM_FIELDS
        acc = plsc.load_gather(g_v, [base + rowpat, colpat])
        for k in range(1, 13):
            acc = acc + plsc.load_gather(g_v, [(base + 2 * k) + rowpat, colpat])
        s_v[r] = acc
        return carry

    lax.fori_loop(0, ROWS_PER_W, red_body, 0)

    for c in range(ROWS_PER_W // 16):
        rows = c * 16 + lane
        tot = plsc.load_gather(s_v, [rows, jnp.zeros((16,), jnp.int32)])
        for col in range(1, 16):
            tot = tot + plsc.load_gather(s_v, [rows, jnp.full((16,), col, jnp.int32)])
        out_v[pl.ds(c * 16, 16)] = 1.0 / (1.0 + jnp.exp(-tot))

    pltpu.sync_copy(out_v, out_hbm.at[pl.ds(wid * ROWS_PER_W, ROWS_PER_W)])


@functools.partial(
    pl.kernel,
    out_type=jax.ShapeDtypeStruct((BATCH,), jnp.float32),
    mesh=plsc.VectorSubcoreMesh(
        core_axis_name="c", subcore_axis_name="s", num_cores=NC, num_subcores=NS
    ),
    scratch_types=[
        pltpu.VMEM((NUM_FIELDS, 128), jnp.int32),
        pltpu.VMEM((IDX_PER_W, EMBED_DIM), jnp.float32),
        pltpu.VMEM((ROWS_PER_W, 16), jnp.float32),
        pltpu.VMEM((ROWS_PER_W,), jnp.float32),
        pltpu.SemaphoreType.DMA,
    ],
    compiler_params=pltpu.CompilerParams(
        use_tc_tiling_on_sc=False, needs_layout_passes=False
    ),
)
def _sc_kernel(idx_hbm, tab_hbm, out_hbm, idx_v, g_v, s_v, out_v, sem):
    _sc_body(idx_hbm, tab_hbm, out_hbm, idx_v, g_v, s_v, out_v, sem)


def kernel(indices, table):
    idx3 = indices.reshape(NW, NUM_FIELDS, 128)
    tab2 = table.reshape(NUM_FIELDS * VOCAB, EMBED_DIM)
    out = _sc_kernel(idx3, tab2)
    return out.reshape(BATCH, 1)


# speedup vs baseline: 5.7471x; 5.7471x over previous
import functools

import jax
import jax.numpy as jnp
from jax import lax
from jax.experimental import pallas as pl
from jax.experimental.pallas import tpu as pltpu
from jax.experimental.pallas import tpu_sc as plsc

NUM_FIELDS = 26
VOCAB = 100000
EMBED_DIM = 8
BATCH = 4096

NC, NS = 2, 16
NW = NC * NS
B_PER_W = BATCH // NW


def _sc_body(idx_hbm, tab_hbm, out_hbm, idx_v, g_v, acc_v, sem):
    cid = lax.axis_index("c")
    sid = lax.axis_index("s")
    wid = sid * NC + cid
    bbase = wid * B_PER_W

    pltpu.sync_copy(idx_hbm.at[:, pl.ds(bbase, B_PER_W)], idx_v)

    def fire(f, buf):
        return [
            pltpu.async_copy(
                tab_hbm.at[pl.ds((f * EMBED_DIM + d) * VOCAB, VOCAB)]
                .at[idx_v.at[f]],
                g_v.at[buf, d],
                sem,
            )
            for d in range(EMBED_DIM)
        ]

    copies = fire(0, 0)
    for f in range(NUM_FIELDS):
        for cp in copies:
            cp.wait()
        if f + 1 < NUM_FIELDS:
            copies = fire(f + 1, (f + 1) % 2)
        buf = f % 2
        for c in range(B_PER_W // 16):
            tot = g_v[buf, 0, pl.ds(c * 16, 16)]
            for d in range(1, EMBED_DIM):
                tot = tot + g_v[buf, d, pl.ds(c * 16, 16)]
            if f == 0:
                acc_v[pl.ds(c * 16, 16)] = tot
            else:
                acc_v[pl.ds(c * 16, 16)] = acc_v[pl.ds(c * 16, 16)] + tot

    for c in range(B_PER_W // 16):
        x = acc_v[pl.ds(c * 16, 16)]
        acc_v[pl.ds(c * 16, 16)] = 1.0 / (1.0 + jnp.exp(-x))
    pltpu.sync_copy(acc_v, out_hbm.at[pl.ds(bbase, B_PER_W)])


@functools.partial(
    pl.kernel,
    out_type=jax.ShapeDtypeStruct((BATCH,), jnp.float32),
    mesh=plsc.VectorSubcoreMesh(
        core_axis_name="c", subcore_axis_name="s", num_cores=NC, num_subcores=NS
    ),
    scratch_types=[
        pltpu.VMEM((NUM_FIELDS, B_PER_W), jnp.int32),
        pltpu.VMEM((2, EMBED_DIM, B_PER_W), jnp.float32),
        pltpu.VMEM((B_PER_W,), jnp.float32),
        pltpu.SemaphoreType.DMA,
    ],
    compiler_params=pltpu.CompilerParams(
        use_tc_tiling_on_sc=False, needs_layout_passes=False
    ),
)
def _sc_kernel(idx_hbm, tab_hbm, out_hbm, idx_v, g_v, acc_v, sem):
    _sc_body(idx_hbm, tab_hbm, out_hbm, idx_v, g_v, acc_v, sem)


def kernel(indices, table):
    idxT = indices.T
    tab_flat = table.transpose(0, 2, 1).reshape(-1)
    out = _sc_kernel(idxT, tab_flat)
    return out.reshape(BATCH, 1)


# speedup vs baseline: 6.1598x; 1.0718x over previous
import functools

import jax
import jax.numpy as jnp
from jax import lax
from jax.experimental import pallas as pl
from jax.experimental.pallas import tpu as pltpu
from jax.experimental.pallas import tpu_sc as plsc

NUM_FIELDS = 26
VOCAB = 100000
EMBED_DIM = 8
BATCH = 4096

NC, NS = 2, 16
NW = NC * NS
B_PER_W = BATCH // NW
NBUF = 4


def _sc_body(idx_hbm, tab_hbm, out_hbm, idx_v, g_v, acc_v, sem):
    cid = lax.axis_index("c")
    sid = lax.axis_index("s")
    wid = sid * NC + cid
    bbase = wid * B_PER_W

    pltpu.sync_copy(idx_hbm.at[:, pl.ds(bbase, B_PER_W)], idx_v)

    def fire(f, buf):
        return [
            pltpu.async_copy(
                tab_hbm.at[pl.ds((f * EMBED_DIM + d) * VOCAB, VOCAB)]
                .at[idx_v.at[f]],
                g_v.at[buf, d],
                sem,
            )
            for d in range(EMBED_DIM)
        ]

    pending = [fire(f, f % NBUF) for f in range(NBUF - 1)]
    for f in range(NUM_FIELDS):
        if f + NBUF - 1 < NUM_FIELDS:
            pending.append(fire(f + NBUF - 1, (f + NBUF - 1) % NBUF))
        for cp in pending.pop(0):
            cp.wait()
        buf = f % NBUF
        for c in range(B_PER_W // 16):
            tot = g_v[buf, 0, pl.ds(c * 16, 16)]
            for d in range(1, EMBED_DIM):
                tot = tot + g_v[buf, d, pl.ds(c * 16, 16)]
            if f == 0:
                acc_v[pl.ds(c * 16, 16)] = tot
            else:
                acc_v[pl.ds(c * 16, 16)] = acc_v[pl.ds(c * 16, 16)] + tot

    for c in range(B_PER_W // 16):
        x = acc_v[pl.ds(c * 16, 16)]
        acc_v[pl.ds(c * 16, 16)] = 1.0 / (1.0 + jnp.exp(-x))
    pltpu.sync_copy(acc_v, out_hbm.at[pl.ds(bbase, B_PER_W)])


@functools.partial(
    pl.kernel,
    out_type=jax.ShapeDtypeStruct((BATCH,), jnp.float32),
    mesh=plsc.VectorSubcoreMesh(
        core_axis_name="c", subcore_axis_name="s", num_cores=NC, num_subcores=NS
    ),
    scratch_types=[
        pltpu.VMEM((NUM_FIELDS, B_PER_W), jnp.int32),
        pltpu.VMEM((NBUF, EMBED_DIM, B_PER_W), jnp.float32),
        pltpu.VMEM((B_PER_W,), jnp.float32),
        pltpu.SemaphoreType.DMA,
    ],
    compiler_params=pltpu.CompilerParams(
        use_tc_tiling_on_sc=False, needs_layout_passes=False
    ),
)
def _sc_kernel(idx_hbm, tab_hbm, out_hbm, idx_v, g_v, acc_v, sem):
    _sc_body(idx_hbm, tab_hbm, out_hbm, idx_v, g_v, acc_v, sem)


def kernel(indices, table):
    idxT = indices.T
    tab_flat = table.transpose(0, 2, 1).reshape(-1)
    out = _sc_kernel(idxT, tab_flat)
    return out.reshape(BATCH, 1)
